# Initial kernel scaffold; baseline (speedup 1.0000x reference)
import jax
import jax.numpy as jnp
from jax.experimental import pallas as pl


def kernel(x, edge_index, dis, train_idx, W_gnn, b_gnn, W_out, b_out):
    raise NotImplementedError("write your pallas kernel here")


# speedup vs baseline: 6.7222x; 6.7222x over previous
import functools

import jax
import jax.numpy as jnp
from jax import lax
from jax.experimental import pallas as pl
from jax.experimental.pallas import tpu as pltpu
from jax.experimental.pallas import tpu_sc as plsc

NC = 2
NS = 16
NW = NC * NS
L = 16

N = 10000
D = 128
E = 320000
P = 131072

EPW = E // NW
EC = 80
ENCH = EPW // EC
NPAD = 10240
NSTRIPE = NPAD // NS

PPW = P // NW
PC = 128
PNCH = PPW // PC


def _matmul_body(x_ref, w_ref, y_ref):
    y_ref[...] = jnp.dot(x_ref[...], w_ref[...],
                         preferred_element_type=jnp.float32)


def _head_body(s2_ref, deg2_ref, bg_ref, wo_ref, bo_ref, u_ref, v_ref):
    s = (s2_ref[0] + s2_ref[1])[:N]
    deg = (deg2_ref[0] + deg2_ref[1])[:N]
    h = s / jnp.maximum(deg, 1.0)[:, None] + bg_ref[...]
    g = jax.nn.sigmoid(h)
    wuv = jnp.concatenate([wo_ref[:D], wo_ref[D:2 * D]], axis=1)
    uv = jnp.dot(g, wuv, preferred_element_type=jnp.float32)
    u_ref[...] = uv[:, 0] + bo_ref[0]
    v_ref[...] = uv[:, 1]


def _edge_body(y_hbm, src_hbm, dst_hbm, z2_hbm, z1_hbm,
               s2_hbm, deg2_hbm,
               src_v, dstm_v, rows_v, ones_v, acc_sh, deg_sh, sem):
    c = lax.axis_index("c")
    s = lax.axis_index("s")
    wid = c * NS + s

    pltpu.sync_copy(src_hbm.at[wid], src_v)
    pltpu.sync_copy(dst_hbm.at[wid], dstm_v)

    for k in range(5):
        pltpu.sync_copy(z2_hbm, acc_sh.at[pl.ds(s * NSTRIPE + k * 128, 128), :])
    pltpu.sync_copy(z1_hbm, deg_sh.at[pl.ds(s * NSTRIPE, NSTRIPE)])

    def _ones(i, _):
        ones_v[pl.ds(i * L, L)] = jnp.ones((L,), jnp.float32)
        return _
    lax.fori_loop(0, EC // L, _ones, None)

    plsc.subcore_barrier()

    def _chunk(j, _):
        pltpu.async_copy(y_hbm.at[src_v.at[pl.ds(j * EC, EC)]], rows_v,
                         sem).wait()
        pltpu.sync_copy(rows_v, acc_sh.at[dstm_v.at[j]], add=True)
        pltpu.sync_copy(ones_v, deg_sh.at[dstm_v.at[j]], add=True)
        return _
    lax.fori_loop(0, ENCH, _chunk, None)

    plsc.subcore_barrier()

    pltpu.sync_copy(acc_sh.at[pl.ds(s * NSTRIPE, NSTRIPE), :],
                    s2_hbm.at[c, pl.ds(s * NSTRIPE, NSTRIPE), :])
    pltpu.sync_copy(deg_sh.at[pl.ds(s * NSTRIPE, NSTRIPE)],
                    deg2_hbm.at[c, pl.ds(s * NSTRIPE, NSTRIPE)])


def _pair_body(u_hbm, v_hbm, ti_hbm, disf_hbm, wd_hbm,
               out_hbm,
               u_v, v_v, o_v, d_v, flat_v, uvsum_v, dis_v, out_v, wd_v, sem):
    c = lax.axis_index("c")
    s = lax.axis_index("s")
    wid = c * NS + s
    base = wid * PPW

    pltpu.sync_copy(u_hbm, u_v)
    pltpu.sync_copy(v_hbm, v_v)
    pltpu.sync_copy(ti_hbm.at[0, pl.ds(base, PPW)], o_v)
    pltpu.sync_copy(ti_hbm.at[1, pl.ds(base, PPW)], d_v)
    pltpu.sync_copy(wd_hbm, wd_v)
    wd = wd_v[...]

    def _uvgather(j, _):
        o16 = o_v[pl.ds(j * L, L)]
        d16 = d_v[pl.ds(j * L, L)]
        flat_v[pl.ds(j * L, L)] = o16 * N + d16
        uo = plsc.load_gather(u_v, [o16])
        vd = plsc.load_gather(v_v, [d16])
        uvsum_v[pl.ds(j * L, L)] = uo + vd
        return _
    lax.fori_loop(0, PPW // L, _uvgather, None)

    def _disgather(j, _):
        pltpu.async_copy(disf_hbm.at[flat_v.at[pl.ds(j * PC, PC)]],
                         dis_v.at[pl.ds(j * PC, PC)], sem).wait()
        return _
    lax.fori_loop(0, PNCH, _disgather, None)

    def _finish(j, _):
        t = uvsum_v[pl.ds(j * L, L)] + wd * dis_v[pl.ds(j * L, L)]
        e = jnp.exp(-2.0 * t)
        out_v[pl.ds(j * L, L)] = 2.0 / (1.0 + e) - 1.0
        return _
    lax.fori_loop(0, PPW // L, _finish, None)

    pltpu.sync_copy(out_v, out_hbm.at[pl.ds(base, PPW)])


@jax.jit
def kernel(x, edge_index, dis, train_idx, W_gnn, b_gnn, W_out, b_out):
    src = edge_index[0].astype(jnp.int32).reshape(NW, EPW)
    dst = edge_index[1].astype(jnp.int32).reshape(NW, ENCH, EC)
    ti = train_idx.astype(jnp.int32)
    disf = dis.reshape(-1)
    z2 = jnp.zeros((128, D), jnp.float32)
    z1 = jnp.zeros((NSTRIPE,), jnp.float32)
    wd_vec = jnp.broadcast_to(W_out[2 * D, 0], (L,))

    y = pl.pallas_call(
        _matmul_body,
        out_shape=jax.ShapeDtypeStruct((N, D), jnp.float32),
    )(x, W_gnn)

    edge_k = pl.kernel(
        _edge_body,
        out_type=(
            jax.ShapeDtypeStruct((NC, NPAD, D), jnp.float32),
            jax.ShapeDtypeStruct((NC, NPAD), jnp.float32),
        ),
        mesh=plsc.VectorSubcoreMesh(core_axis_name="c", subcore_axis_name="s"),
        scratch_types=[
            pltpu.VMEM((EPW,), jnp.int32),
            pltpu.VMEM((ENCH, EC), jnp.int32),
            pltpu.VMEM((EC, D), jnp.float32),
            pltpu.VMEM((EC,), jnp.float32),
            pltpu.VMEM_SHARED((NPAD, D), jnp.float32),
            pltpu.VMEM_SHARED((NPAD,), jnp.float32),
            pltpu.SemaphoreType.DMA,
        ],
    )
    s2, deg2 = edge_k(y, src, dst, z2, z1)

    u, v = pl.pallas_call(
        _head_body,
        out_shape=(jax.ShapeDtypeStruct((N,), jnp.float32),
                   jax.ShapeDtypeStruct((N,), jnp.float32)),
    )(s2, deg2, b_gnn, W_out, b_out)

    pair_k = pl.kernel(
        _pair_body,
        out_type=jax.ShapeDtypeStruct((P,), jnp.float32),
        mesh=plsc.VectorSubcoreMesh(core_axis_name="c", subcore_axis_name="s"),
        compiler_params=pltpu.CompilerParams(needs_layout_passes=False),
        scratch_types=[
            pltpu.VMEM((N,), jnp.float32),
            pltpu.VMEM((N,), jnp.float32),
            pltpu.VMEM((PPW,), jnp.int32),
            pltpu.VMEM((PPW,), jnp.int32),
            pltpu.VMEM((PPW,), jnp.int32),
            pltpu.VMEM((PPW,), jnp.float32),
            pltpu.VMEM((PPW,), jnp.float32),
            pltpu.VMEM((PPW,), jnp.float32),
            pltpu.VMEM((L,), jnp.float32),
            pltpu.SemaphoreType.DMA,
        ],
    )
    pre = pair_k(u, v, ti, disf, wd_vec)
    return pre.reshape(P, 1)


# speedup vs baseline: 6.9205x; 1.0295x over previous
import jax
import jax.numpy as jnp
from jax import lax
from jax.experimental import pallas as pl
from jax.experimental.pallas import tpu as pltpu
from jax.experimental.pallas import tpu_sc as plsc

NC = 2
NS = 16
NW = NC * NS
L = 16

N = 10000
D = 128
E = 320000
P = 131072

EPW = E // NW
EC = 80
ENCH = EPW // EC
NPAD = 10240
NSTRIPE = NPAD // NS
RB = 2
HR = NPAD // D

PPW = P // NW
PC = 128
PNCH = PPW // PC


def _matmul_body(x_ref, w_ref, y_ref):
    y_ref[...] = jnp.dot(x_ref[...], w_ref[...],
                         preferred_element_type=jnp.float32)


def _head_body(s2_ref, degw_ref, bg_ref, wo_ref, bo_ref, u_ref, v_ref):
    s = (s2_ref[0] + s2_ref[1])[:N]
    deg = jnp.sum(degw_ref[...].reshape(NW, NPAD), axis=0)[:N]
    h = s / jnp.maximum(deg, 1.0)[:, None] + bg_ref[...]
    g = jax.nn.sigmoid(h)
    wuv = jnp.concatenate([wo_ref[:D], wo_ref[D:2 * D]], axis=1)
    uv = jnp.dot(g, wuv, preferred_element_type=jnp.float32)
    u_ref[...] = uv[:, 0] + bo_ref[0]
    v_ref[...] = uv[:, 1]


def _deg_body(dst_hbm, z2_hbm, degw_hbm, dstm_v, hist2):
    c = lax.axis_index("c")
    s = lax.axis_index("s")
    wid = c * NS + s

    pltpu.sync_copy(dst_hbm.at[wid], dstm_v)
    pltpu.sync_copy(z2_hbm.at[pl.ds(0, HR)], hist2)

    ones16 = jnp.ones((L,), jnp.float32)

    def _hist(j, _):
        for k in range(EC // L):
            d16 = dstm_v[j, pl.ds(k * L, L)]
            row16 = lax.shift_right_logical(d16, 7)
            col16 = lax.bitwise_and(d16, jnp.int32(127))
            plsc.addupdate_scatter(hist2, [row16, col16], ones16)
        return _
    lax.fori_loop(0, ENCH, _hist, None)

    pltpu.sync_copy(hist2, degw_hbm.at[wid])


def _edge_body(y_hbm, src_hbm, dst_hbm, z2_hbm,
               s2_hbm,
               src_v, dstm_v, rows_v, acc_sh, gsem):
    c = lax.axis_index("c")
    s = lax.axis_index("s")
    wid = c * NS + s

    pltpu.sync_copy(src_hbm.at[wid], src_v)
    pltpu.sync_copy(dst_hbm.at[wid], dstm_v)

    for k in range(5):
        pltpu.sync_copy(z2_hbm, acc_sh.at[pl.ds(s * NSTRIPE + k * 128, 128), :])

    plsc.subcore_barrier()

    def _gfire(j, b):
        pltpu.async_copy(y_hbm.at[src_v.at[pl.ds(j * EC, EC)]],
                         rows_v.at[b], gsem)

    def _consume(j, b):
        pltpu.make_async_copy(y_hbm.at[src_v.at[pl.ds(j * EC, EC)]],
                              rows_v.at[b], gsem).wait()
        pltpu.sync_copy(rows_v.at[b], acc_sh.at[dstm_v.at[j]], add=True)

    for b in range(RB):
        _gfire(b, b)

    def _main(j, _):
        b = lax.rem(j, RB)
        _consume(j, b)
        _gfire(j + RB, b)
        return _
    lax.fori_loop(0, ENCH - RB, _main, None)

    for j in range(ENCH - RB, ENCH):
        _consume(j, j % RB)

    plsc.subcore_barrier()

    pltpu.sync_copy(acc_sh.at[pl.ds(s * NSTRIPE, NSTRIPE), :],
                    s2_hbm.at[c, pl.ds(s * NSTRIPE, NSTRIPE), :])


def _pair_body(u_hbm, v_hbm, ti_hbm, disf_hbm, wd_hbm,
               out_hbm,
               u_v, v_v, o_v, d_v, flat_v, uvsum_v, dis_v, out_v, wd_v, sem):
    c = lax.axis_index("c")
    s = lax.axis_index("s")
    wid = c * NS + s
    base = wid * PPW

    pltpu.sync_copy(ti_hbm.at[0, pl.ds(base, PPW)], o_v)
    pltpu.sync_copy(ti_hbm.at[1, pl.ds(base, PPW)], d_v)

    def _flat(j, _):
        o16 = o_v[pl.ds(j * L, L)]
        d16 = d_v[pl.ds(j * L, L)]
        flat_v[pl.ds(j * L, L)] = o16 * N + d16
        return _
    lax.fori_loop(0, PPW // L, _flat, None)

    def _disfire(j, _):
        pltpu.async_copy(disf_hbm.at[flat_v.at[pl.ds(j * PC, PC)]],
                         dis_v.at[pl.ds(j * PC, PC)], sem)
        return _
    lax.fori_loop(0, PNCH, _disfire, None)

    pltpu.sync_copy(u_hbm, u_v)
    pltpu.sync_copy(v_hbm, v_v)
    pltpu.sync_copy(wd_hbm, wd_v)
    wd = wd_v[...]

    def _uvgather(j, _):
        o16 = o_v[pl.ds(j * L, L)]
        d16 = d_v[pl.ds(j * L, L)]
        uo = plsc.load_gather(u_v, [o16])
        vd = plsc.load_gather(v_v, [d16])
        uvsum_v[pl.ds(j * L, L)] = uo + vd
        return _
    lax.fori_loop(0, PPW // L, _uvgather, None)

    def _disdrain(j, _):
        pltpu.make_async_copy(disf_hbm.at[flat_v.at[pl.ds(j * PC, PC)]],
                              dis_v.at[pl.ds(j * PC, PC)], sem).wait()
        return _
    lax.fori_loop(0, PNCH, _disdrain, None)

    def _finish(j, _):
        t = uvsum_v[pl.ds(j * L, L)] + wd * dis_v[pl.ds(j * L, L)]
        e = jnp.exp(-2.0 * t)
        out_v[pl.ds(j * L, L)] = 2.0 / (1.0 + e) - 1.0
        return _
    lax.fori_loop(0, PPW // L, _finish, None)

    pltpu.sync_copy(out_v, out_hbm.at[pl.ds(base, PPW)])


@jax.jit
def kernel(x, edge_index, dis, train_idx, W_gnn, b_gnn, W_out, b_out):
    src = edge_index[0].astype(jnp.int32).reshape(NW, EPW)
    dstc = edge_index[1].astype(jnp.int32).reshape(NW, ENCH, EC)
    ti = train_idx.astype(jnp.int32)
    disf = dis.reshape(-1)
    z2 = jnp.zeros((128, D), jnp.float32)
    wd_vec = jnp.broadcast_to(W_out[2 * D, 0], (L,))

    deg_k = pl.kernel(
        _deg_body,
        out_type=jax.ShapeDtypeStruct((NW, HR, D), jnp.float32),
        mesh=plsc.VectorSubcoreMesh(core_axis_name="c", subcore_axis_name="s"),
        compiler_params=pltpu.CompilerParams(needs_layout_passes=False),
        scratch_types=[
            pltpu.VMEM((ENCH, EC), jnp.int32),
            pltpu.VMEM((HR, D), jnp.float32),
        ],
    )
    degw = deg_k(dstc, z2)

    y = pl.pallas_call(
        _matmul_body,
        out_shape=jax.ShapeDtypeStruct((N, D), jnp.float32),
    )(x, W_gnn)

    edge_k = pl.kernel(
        _edge_body,
        out_type=jax.ShapeDtypeStruct((NC, NPAD, D), jnp.float32),
        mesh=plsc.VectorSubcoreMesh(core_axis_name="c", subcore_axis_name="s"),
        scratch_types=[
            pltpu.VMEM((EPW,), jnp.int32),
            pltpu.VMEM((ENCH, EC), jnp.int32),
            pltpu.VMEM((RB, EC, D), jnp.float32),
            pltpu.VMEM_SHARED((NPAD, D), jnp.float32),
            pltpu.SemaphoreType.DMA,
        ],
    )
    s2 = edge_k(y, src, dstc, z2)

    u, v = pl.pallas_call(
        _head_body,
        out_shape=(jax.ShapeDtypeStruct((N,), jnp.float32),
                   jax.ShapeDtypeStruct((N,), jnp.float32)),
    )(s2, degw, b_gnn, W_out, b_out)

    pair_k = pl.kernel(
        _pair_body,
        out_type=jax.ShapeDtypeStruct((P,), jnp.float32),
        mesh=plsc.VectorSubcoreMesh(core_axis_name="c", subcore_axis_name="s"),
        compiler_params=pltpu.CompilerParams(needs_layout_passes=False),
        scratch_types=[
            pltpu.VMEM((N,), jnp.float32),
            pltpu.VMEM((N,), jnp.float32),
            pltpu.VMEM((PPW,), jnp.int32),
            pltpu.VMEM((PPW,), jnp.int32),
            pltpu.VMEM((PPW,), jnp.int32),
            pltpu.VMEM((PPW,), jnp.float32),
            pltpu.VMEM((PPW,), jnp.float32),
            pltpu.VMEM((PPW,), jnp.float32),
            pltpu.VMEM((L,), jnp.float32),
            pltpu.SemaphoreType.DMA,
        ],
    )
    pre = pair_k(u, v, ti, disf, wd_vec)
    return pre.reshape(P, 1)
